# Initial kernel scaffold; baseline (speedup 1.0000x reference)
import jax
import jax.numpy as jnp
from jax.experimental import pallas as pl


def kernel(tokens, key_ids, write_pos, query_pos, value_ids, embed_W, key_embed_W, value_embed_W, W1, b1, W2, b2, Wx, Uh, bl, tau_raw, h0, z2v_W, z2v_b, logit_scale_raw, slotA):
    raise NotImplementedError("write your pallas kernel here")


# speedup vs baseline: 21.4091x; 21.4091x over previous
import functools

import jax
import jax.numpy as jnp
from jax import lax
from jax.experimental import pallas as pl
from jax.experimental.pallas import tpu as pltpu
from jax.experimental.pallas import tpu_sc as plsc

B, L = 1024, 32
VOCAB = 100000
EMB, HID, MEM = 64, 128, 64
NV = 1000
NSLOTS, KTOP = 256, 16
MAXIT, RELAX, DECAY = 8, 0.5, 0.997
HIDDEN = 256
TW = 16

NC, NS = 2, 16
NW = NC * NS
TPW = TW * B // NW
KPW = B // NW
CHUNK = 128

def _softplus(x):
    return jnp.where(x > 0, x + jnp.log1p(jnp.exp(-x)), jnp.log1p(jnp.exp(x)))


@functools.cache
def _sc_kernels():
    mesh = plsc.VectorSubcoreMesh(core_axis_name="c", subcore_axis_name="s",
                                  num_cores=NC, num_subcores=NS)

    @functools.partial(
        pl.kernel,
        out_type=(jax.ShapeDtypeStruct((TW * B, EMB), jnp.float32),
                  jax.ShapeDtypeStruct((B, MEM), jnp.float32)),
        mesh=mesh,
        compiler_params=pltpu.CompilerParams(use_tc_tiling_on_sc=False),
        scratch_types=(pltpu.VMEM((TPW // CHUNK, CHUNK), jnp.int32),
                       pltpu.VMEM((TPW, EMB), jnp.float32),
                       pltpu.VMEM((KPW,), jnp.int32),
                       pltpu.VMEM((KPW, MEM), jnp.float32),
                       pltpu.SemaphoreType.DMA),
    )
    def _embed_gather(emb_hbm, tok_hbm, keyw_hbm, kid_hbm, x_out, kv_out,
                      tok_v, xr_v, kid_v, kr_v, sem):
        wid = lax.axis_index("s") * NC + lax.axis_index("c")
        pltpu.sync_copy(tok_hbm.at[wid], tok_v)
        for j in range(TPW // CHUNK):
            pltpu.async_copy(emb_hbm.at[tok_v.at[j]],
                             xr_v.at[pl.ds(j * CHUNK, CHUNK)], sem).wait()
        pltpu.sync_copy(xr_v, x_out.at[pl.ds(wid * TPW, TPW)])
        pltpu.sync_copy(kid_hbm.at[pl.ds(wid * KPW, KPW)], kid_v)
        pltpu.async_copy(keyw_hbm.at[kid_v], kr_v, sem).wait()
        pltpu.sync_copy(kr_v, kv_out.at[pl.ds(wid * KPW, KPW)])

    @functools.partial(
        pl.kernel,
        out_type=jax.ShapeDtypeStruct((NC, TW * NSLOTS, MEM), jnp.float32),
        mesh=mesh,
        compiler_params=pltpu.CompilerParams(use_tc_tiling_on_sc=False),
        scratch_types=(pltpu.VMEM((TPW // CHUNK, CHUNK), jnp.int32),
                       pltpu.VMEM((TPW, MEM), jnp.float32),
                       pltpu.VMEM_SHARED((TW * NSLOTS, MEM), jnp.float32),
                       pltpu.SemaphoreType.DMA),
    )
    def _scatter(rows_hbm, idx_hbm, zeros_hbm, s_out, idx_v, rows_v, shared,
                 sem):
        cid = lax.axis_index("c")
        sid = lax.axis_index("s")
        wid = sid * NC + cid
        pltpu.sync_copy(zeros_hbm, shared.at[pl.ds(sid * NSLOTS, NSLOTS)])
        plsc.subcore_barrier()
        pltpu.sync_copy(idx_hbm.at[wid], idx_v)
        pltpu.sync_copy(rows_hbm.at[pl.ds(wid * TPW, TPW)], rows_v)
        for j in range(TPW // CHUNK):
            pltpu.sync_copy(rows_v.at[pl.ds(j * CHUNK, CHUNK)],
                            shared.at[idx_v.at[j]], add=True)
        plsc.subcore_barrier()
        pltpu.sync_copy(shared.at[pl.ds(sid * NSLOTS, NSLOTS)],
                        s_out.at[cid, pl.ds(sid * NSLOTS, NSLOTS)])

    @functools.partial(
        pl.kernel,
        out_type=jax.ShapeDtypeStruct((KTOP * B, MEM), jnp.float32),
        mesh=mesh,
        compiler_params=pltpu.CompilerParams(use_tc_tiling_on_sc=False),
        scratch_types=(pltpu.VMEM((TPW // CHUNK, CHUNK), jnp.int32),
                       pltpu.VMEM((TPW, MEM), jnp.float32),
                       pltpu.SemaphoreType.DMA),
    )
    def _qgather(m_hbm, qidx_hbm, out, idx_v, rows_v, sem):
        wid = lax.axis_index("s") * NC + lax.axis_index("c")
        pltpu.sync_copy(qidx_hbm.at[wid], idx_v)
        for j in range(TPW // CHUNK):
            pltpu.async_copy(m_hbm.at[idx_v.at[j]],
                             rows_v.at[pl.ds(j * CHUNK, CHUNK)], sem).wait()
        pltpu.sync_copy(rows_v, out.at[pl.ds(wid * TPW, TPW)])

    return _embed_gather, _scatter, _qgather


def _mega_body(x_ref, kv_ref, wp_ref, qp_ref, w1_ref, b1_ref, w2_ref, b2_ref,
               wx_ref, uh_ref, bl_ref, tau_ref, h0_ref, z2v_ref, z2vb_ref,
               slota_ref, wv_ref, widx_ref, qidx_ref, w_ref, d_ref):
    f32 = jnp.float32

    def dot(a, b):
        return lax.dot_general(a, b, (((1,), (0,)), ((), ())),
                               preferred_element_type=f32)

    def dot_t(a, b):
        return lax.dot_general(a, b, (((1,), (1,)), ((), ())),
                               preferred_element_type=f32)

    kv = kv_ref[...]
    kv = kv / jnp.maximum(jnp.sqrt(jnp.sum(kv * kv, axis=1, keepdims=True)),
                          1e-12)
    sim = dot_t(kv, slota_ref[...])

    iota_s = lax.broadcasted_iota(jnp.int32, (B, NSLOTS), 1)
    simm = sim
    tv, ti = [], []
    for _ in range(KTOP):
        m = jnp.max(simm, axis=1, keepdims=True)
        idx = jnp.min(jnp.where(simm == m, iota_s, NSLOTS), axis=1,
                      keepdims=True)
        tv.append(m)
        ti.append(idx)
        simm = jnp.where(iota_s == idx, -jnp.inf, simm)
    topv = jnp.concatenate(tv, axis=1)
    topi = jnp.concatenate(ti, axis=1)
    e = jnp.exp(topv - topv[:, 0:1])
    w = e / jnp.sum(e, axis=1, keepdims=True)

    w1a = w1_ref[0:HID, :]
    w1b = w1_ref[HID:HID + EMB, :]
    b1 = b1_ref[...]
    w2 = w2_ref[...]
    b2 = b2_ref[...]
    wx = wx_ref[...]
    uh = uh_ref[...]
    bl = bl_ref[...]
    tau = _softplus(tau_ref[...]) + 1.0
    z1 = z2v_ref[0:HID, :]
    z2 = z2v_ref[HID:2 * HID, :]
    z2vb = z2vb_ref[...]
    wp = wp_ref[...]

    z = jnp.zeros((B, HID), f32)
    h = jnp.broadcast_to(h0_ref[...], (B, HID))
    vw = jnp.zeros((B, MEM), f32)
    for t in range(TW):
        x_t = x_ref[t]
        xw = dot(x_t, w1b) + b1

        def body(i, zz, xw=xw):
            f = zz + dot(jnp.tanh(dot(zz, w1a) + xw), w2) + b2
            return (1.0 - RELAX) * zz + RELAX * f

        z = lax.fori_loop(0, MAXIT, body, z)
        pre = jnp.tanh(dot(z, wx) + dot(h, uh) + bl)
        h = h + (pre - h) / tau
        v = dot(z, z1) + dot(h, z2) + z2vb
        v = v / jnp.maximum(jnp.sqrt(jnp.sum(v * v, axis=1, keepdims=True)),
                            1e-12)
        vw = jnp.where(wp == t, v, vw)

    w_ref[...] = w
    widx_ref[...] = wp * NSLOTS + topi
    qidx_ref[...] = jnp.minimum(qp_ref[...], TW - 1) * NSLOTS + topi
    for k in range(KTOP):
        wv_ref[k] = w[:, k:k + 1] * vw
    iota_t = lax.broadcasted_iota(jnp.int32, (B, TW), 1)
    cnt = jnp.sum((wp == iota_t).astype(f32), axis=0, keepdims=True)
    d_ref[...] = jnp.where(cnt > 0, f32(DECAY), f32(1.0))


def _mstack_body(s_ref, d_ref, out_ref):
    s = s_ref[0] + s_ref[1]
    dv = d_ref[...]
    m = jnp.zeros((NSLOTS, MEM), jnp.float32)
    for t in range(TW):
        m = dv[0:1, t:t + 1] * m + s[t]
        out_ref[t] = m


def _final_body(m_ref, w_ref, vemb_ref, lsr_ref, out_ref):
    w = w_ref[...]
    vm = jnp.zeros((B, MEM), jnp.float32)
    for k in range(KTOP):
        vm = vm + w[:, k:k + 1] * m_ref[k]
    vm = vm / jnp.maximum(jnp.sqrt(jnp.sum(vm * vm, axis=1, keepdims=True)),
                          1e-12)
    vp = vemb_ref[...]
    vp = vp / jnp.maximum(jnp.sqrt(jnp.sum(vp * vp, axis=1, keepdims=True)),
                          1e-12)
    scale = _softplus(lsr_ref[...]) + 1e-3
    out_ref[...] = scale * lax.dot_general(
        vm, vp, (((1,), (1,)), ((), ())), preferred_element_type=jnp.float32)


def kernel(tokens, key_ids, write_pos, query_pos, value_ids, embed_W,
           key_embed_W, value_embed_W, W1, b1, W2, b2, Wx, Uh, bl, tau_raw,
           h0, z2v_W, z2v_b, logit_scale_raw, slotA):
    i32 = jnp.int32
    f32 = jnp.float32
    embed_gather, scatter, qgather = _sc_kernels()
    tok3 = tokens[:, :TW].astype(i32).T.reshape(NW, TPW // CHUNK, CHUNK)
    x_rows, kv_raw = embed_gather(embed_W, tok3, key_embed_W,
                                  key_ids.astype(i32))
    x3 = x_rows.reshape(TW, B, EMB)

    mega = pl.pallas_call(_mega_body, out_shape=(
        jax.ShapeDtypeStruct((KTOP, B, MEM), f32),
        jax.ShapeDtypeStruct((B, KTOP), i32),
        jax.ShapeDtypeStruct((B, KTOP), i32),
        jax.ShapeDtypeStruct((B, KTOP), f32),
        jax.ShapeDtypeStruct((1, TW), f32),
    ))
    wv, widx, qidx, wts, dvec = mega(
        x3, kv_raw, write_pos.astype(i32).reshape(B, 1),
        query_pos.astype(i32).reshape(B, 1), W1, b1.reshape(1, -1), W2,
        b2.reshape(1, -1), Wx, Uh, bl.reshape(1, -1), tau_raw.reshape(1, -1),
        h0.reshape(1, -1), z2v_W, z2v_b.reshape(1, -1), slotA)

    wv_rows = wv.reshape(KTOP * B, MEM)
    widx3 = widx.T.reshape(NW, TPW // CHUNK, CHUNK)
    zeros = jnp.zeros((NSLOTS, MEM), f32)
    s2 = scatter(wv_rows, widx3, zeros)

    mstack = pl.pallas_call(_mstack_body, out_shape=jax.ShapeDtypeStruct(
        (TW, NSLOTS, MEM), f32))
    mflat = mstack(s2.reshape(NC, TW, NSLOTS, MEM), dvec)

    qidx3 = qidx.T.reshape(NW, TPW // CHUNK, CHUNK)
    mrows = qgather(mflat.reshape(TW * NSLOTS, MEM), qidx3)

    final = pl.pallas_call(_final_body, out_shape=jax.ShapeDtypeStruct(
        (B, NV), f32))
    return final(mrows.reshape(KTOP, B, MEM), wts, value_embed_W,
                 logit_scale_raw.reshape(1, 1))


# speedup vs baseline: 23.1385x; 1.0808x over previous
import functools

import jax
import jax.numpy as jnp
from jax import lax
from jax.experimental import pallas as pl
from jax.experimental.pallas import tpu as pltpu
from jax.experimental.pallas import tpu_sc as plsc

B, L = 1024, 32
VOCAB = 100000
EMB, HID, MEM = 64, 128, 64
NV = 1000
NSLOTS, KTOP = 256, 16
MAXIT, RELAX, DECAY = 8, 0.5, 0.997
HIDDEN = 256
TW = 16
PADW = 128

NC, NS = 2, 16
NW = NC * NS
TPW = TW * B // NW
KPW = B // NW
CHUNK = 128
NCH = TPW // CHUNK


def _softplus(x):
    return jnp.where(x > 0, x + jnp.log1p(jnp.exp(-x)), jnp.log1p(jnp.exp(x)))


@functools.cache
def _sc_kernels():
    mesh = plsc.VectorSubcoreMesh(core_axis_name="c", subcore_axis_name="s",
                                  num_cores=NC, num_subcores=NS)

    @functools.partial(
        pl.kernel,
        out_type=(jax.ShapeDtypeStruct((TW * B, PADW), jnp.float32),
                  jax.ShapeDtypeStruct((B, PADW), jnp.float32)),
        mesh=mesh,
        compiler_params=pltpu.CompilerParams(use_tc_tiling_on_sc=False),
        scratch_types=(pltpu.VMEM((NCH, CHUNK), jnp.int32),
                       pltpu.VMEM((TPW, EMB), jnp.float32),
                       pltpu.VMEM((KPW,), jnp.int32),
                       pltpu.VMEM((KPW, MEM), jnp.float32),
                       pltpu.SemaphoreType.DMA),
    )
    def _embed_gather(emb_hbm, tok_hbm, keyw_hbm, kid_hbm, x_out, kv_out,
                      tok_v, xr_v, kid_v, kr_v, sem):
        wid = lax.axis_index("s") * NC + lax.axis_index("c")
        pltpu.sync_copy(tok_hbm.at[pl.ds(wid * NCH, NCH)], tok_v)
        for j in range(NCH):
            pltpu.async_copy(emb_hbm.at[tok_v.at[j]],
                             xr_v.at[pl.ds(j * CHUNK, CHUNK)], sem).wait()
        pltpu.sync_copy(xr_v, x_out.at[pl.ds(wid * TPW, TPW), pl.ds(0, EMB)])
        pltpu.sync_copy(kid_hbm.at[pl.ds(wid * KPW, KPW)], kid_v)
        pltpu.async_copy(keyw_hbm.at[kid_v], kr_v, sem).wait()
        pltpu.sync_copy(kr_v, kv_out.at[pl.ds(wid * KPW, KPW), pl.ds(0, MEM)])

    @functools.partial(
        pl.kernel,
        out_type=jax.ShapeDtypeStruct((NC, TW * NSLOTS, PADW), jnp.float32),
        mesh=mesh,
        compiler_params=pltpu.CompilerParams(use_tc_tiling_on_sc=False),
        scratch_types=(pltpu.VMEM((NCH, CHUNK), jnp.int32),
                       pltpu.VMEM((TPW, PADW), jnp.float32),
                       pltpu.VMEM_SHARED((TW * NSLOTS, PADW), jnp.float32),
                       pltpu.SemaphoreType.DMA),
    )
    def _scatter(rows_hbm, idx_hbm, zeros_hbm, s_out, idx_v, rows_v, shared,
                 sem):
        cid = lax.axis_index("c")
        sid = lax.axis_index("s")
        wid = sid * NC + cid
        pltpu.sync_copy(zeros_hbm, shared.at[pl.ds(sid * NSLOTS, NSLOTS)])
        plsc.subcore_barrier()
        pltpu.sync_copy(idx_hbm.at[pl.ds(wid * NCH, NCH)], idx_v)
        pltpu.sync_copy(rows_hbm.at[pl.ds(wid * TPW, TPW)], rows_v)
        for j in range(NCH):
            pltpu.sync_copy(rows_v.at[pl.ds(j * CHUNK, CHUNK)],
                            shared.at[idx_v.at[j]], add=True)
        plsc.subcore_barrier()
        pltpu.sync_copy(shared.at[pl.ds(sid * NSLOTS, NSLOTS)],
                        s_out.at[cid, pl.ds(sid * NSLOTS, NSLOTS)])

    @functools.partial(
        pl.kernel,
        out_type=jax.ShapeDtypeStruct((KTOP * B, PADW), jnp.float32),
        mesh=mesh,
        compiler_params=pltpu.CompilerParams(use_tc_tiling_on_sc=False),
        scratch_types=(pltpu.VMEM((NCH, CHUNK), jnp.int32),
                       pltpu.VMEM((TPW, PADW), jnp.float32),
                       pltpu.SemaphoreType.DMA),
    )
    def _qgather(m_hbm, qidx_hbm, out, idx_v, rows_v, sem):
        wid = lax.axis_index("s") * NC + lax.axis_index("c")
        pltpu.sync_copy(qidx_hbm.at[pl.ds(wid * NCH, NCH)], idx_v)
        for j in range(NCH):
            pltpu.async_copy(m_hbm.at[idx_v.at[j]],
                             rows_v.at[pl.ds(j * CHUNK, CHUNK)], sem).wait()
        pltpu.sync_copy(rows_v, out.at[pl.ds(wid * TPW, TPW)])

    return _embed_gather, _scatter, _qgather


def _mega_body(x_ref, kv_ref, wp_ref, qp_ref, w1_ref, b1_ref, w2_ref, b2_ref,
               wx_ref, uh_ref, bl_ref, tau_ref, h0_ref, z2v_ref, z2vb_ref,
               slota_ref, wv_ref, widx_ref, qidx_ref, w_ref, d_ref):
    f32 = jnp.float32

    def dot(a, b):
        return lax.dot_general(a, b, (((1,), (0,)), ((), ())),
                               preferred_element_type=f32)

    def dot_t(a, b):
        return lax.dot_general(a, b, (((1,), (1,)), ((), ())),
                               preferred_element_type=f32)

    kv = kv_ref[:, 0:MEM]
    kv = kv / jnp.maximum(jnp.sqrt(jnp.sum(kv * kv, axis=1, keepdims=True)),
                          1e-12)
    sim = dot_t(kv, slota_ref[...])

    iota_s = lax.broadcasted_iota(jnp.int32, (B, NSLOTS), 1)
    simm = sim
    tv, ti = [], []
    for _ in range(KTOP):
        m = jnp.max(simm, axis=1, keepdims=True)
        idx = jnp.min(jnp.where(simm == m, iota_s, NSLOTS), axis=1,
                      keepdims=True)
        tv.append(m)
        ti.append(idx)
        simm = jnp.where(iota_s == idx, -jnp.inf, simm)
    topv = jnp.concatenate(tv, axis=1)
    topi = jnp.concatenate(ti, axis=1)
    e = jnp.exp(topv - topv[:, 0:1])
    w = e / jnp.sum(e, axis=1, keepdims=True)

    w1a = w1_ref[0:HID, :]
    w1b = w1_ref[HID:HID + EMB, :]
    b1 = b1_ref[...]
    w2 = w2_ref[...]
    b2 = b2_ref[...]
    wx = wx_ref[...]
    uh = uh_ref[...]
    bl = bl_ref[...]
    tau = _softplus(tau_ref[...]) + 1.0
    z1 = z2v_ref[0:HID, :]
    z2 = z2v_ref[HID:2 * HID, :]
    z2vb = z2vb_ref[...]
    wp = wp_ref[...]

    z = jnp.zeros((B, HID), f32)
    h = jnp.broadcast_to(h0_ref[...], (B, HID))
    vw = jnp.zeros((B, MEM), f32)
    for t in range(TW):
        x_t = x_ref[t, :, 0:EMB]
        xw = dot(x_t, w1b) + b1

        def body(i, zz, xw=xw):
            f = zz + dot(jnp.tanh(dot(zz, w1a) + xw), w2) + b2
            return (1.0 - RELAX) * zz + RELAX * f

        z = lax.fori_loop(0, MAXIT, body, z)
        pre = jnp.tanh(dot(z, wx) + dot(h, uh) + bl)
        h = h + (pre - h) / tau
        v = dot(z, z1) + dot(h, z2) + z2vb
        v = v / jnp.maximum(jnp.sqrt(jnp.sum(v * v, axis=1, keepdims=True)),
                            1e-12)
        vw = jnp.where(wp == t, v, vw)

    w_ref[...] = jnp.transpose(w)
    widx_ref[...] = jnp.transpose(wp * NSLOTS + topi)
    qidx_ref[...] = jnp.transpose(
        jnp.minimum(qp_ref[...], TW - 1) * NSLOTS + topi)
    pad = jnp.zeros((B, PADW - MEM), f32)
    for k in range(KTOP):
        wv_ref[k] = jnp.concatenate([w[:, k:k + 1] * vw, pad], axis=1)
    iota_t = lax.broadcasted_iota(jnp.int32, (B, TW), 1)
    cnt = jnp.sum((wp == iota_t).astype(f32), axis=0, keepdims=True)
    dd = jnp.where(cnt > 0, f32(DECAY), f32(1.0))
    d_ref[...] = jnp.pad(dd, ((0, 7), (0, PADW - TW)))


def _mstack_body(s_ref, d_ref, out_ref):
    s = s_ref[0] + s_ref[1]
    dv = d_ref[0:1, 0:TW]
    m = jnp.zeros((NSLOTS, PADW), jnp.float32)
    for t in range(TW):
        m = dv[0:1, t:t + 1] * m + s[t]
        out_ref[t] = m


def _final_body(m_ref, w_ref, vemb_ref, lsr_ref, out_ref):
    w = jnp.transpose(w_ref[...])
    vm = jnp.zeros((B, MEM), jnp.float32)
    for k in range(KTOP):
        vm = vm + w[:, k:k + 1] * m_ref[k, :, 0:MEM]
    vm = vm / jnp.maximum(jnp.sqrt(jnp.sum(vm * vm, axis=1, keepdims=True)),
                          1e-12)
    vp = vemb_ref[...]
    vp = vp / jnp.maximum(jnp.sqrt(jnp.sum(vp * vp, axis=1, keepdims=True)),
                          1e-12)
    scale = _softplus(lsr_ref[...]) + 1e-3
    out_ref[...] = scale * lax.dot_general(
        vm, vp, (((1,), (1,)), ((), ())), preferred_element_type=jnp.float32)


def kernel(tokens, key_ids, write_pos, query_pos, value_ids, embed_W,
           key_embed_W, value_embed_W, W1, b1, W2, b2, Wx, Uh, bl, tau_raw,
           h0, z2v_W, z2v_b, logit_scale_raw, slotA):
    i32 = jnp.int32
    f32 = jnp.float32
    embed_gather, scatter, qgather = _sc_kernels()
    tok2d = tokens[:, :TW].astype(i32).T.reshape(NW * NCH, CHUNK)
    x_rows, kv_pad = embed_gather(embed_W, tok2d, key_embed_W,
                                  key_ids.astype(i32))
    x3 = x_rows.reshape(TW, B, PADW)

    mega = pl.pallas_call(_mega_body, out_shape=(
        jax.ShapeDtypeStruct((KTOP, B, PADW), f32),
        jax.ShapeDtypeStruct((KTOP, B), i32),
        jax.ShapeDtypeStruct((KTOP, B), i32),
        jax.ShapeDtypeStruct((KTOP, B), f32),
        jax.ShapeDtypeStruct((8, PADW), f32),
    ))
    wv, widx, qidx, wts, dpad = mega(
        x3, kv_pad, write_pos.astype(i32).reshape(B, 1),
        query_pos.astype(i32).reshape(B, 1), W1, b1.reshape(1, -1), W2,
        b2.reshape(1, -1), Wx, Uh, bl.reshape(1, -1), tau_raw.reshape(1, -1),
        h0.reshape(1, -1), z2v_W, z2v_b.reshape(1, -1), slotA)

    wv_rows = wv.reshape(KTOP * B, PADW)
    widx2d = widx.reshape(NW * NCH, CHUNK)
    zeros = jnp.zeros((NSLOTS, PADW), f32)
    s2 = scatter(wv_rows, widx2d, zeros)

    mstack = pl.pallas_call(_mstack_body, out_shape=jax.ShapeDtypeStruct(
        (TW, NSLOTS, PADW), f32))
    mflat = mstack(s2.reshape(NC, TW, NSLOTS, PADW), dpad)

    qidx2d = qidx.reshape(NW * NCH, CHUNK)
    mrows = qgather(mflat.reshape(TW * NSLOTS, PADW), qidx2d)

    final = pl.pallas_call(_final_body, out_shape=jax.ShapeDtypeStruct(
        (B, NV), f32))
    return final(mrows.reshape(KTOP, B, PADW), wts, value_embed_W,
                 logit_scale_raw.reshape(1, 1))


# speedup vs baseline: 23.7491x; 1.0264x over previous
import functools

import jax
import jax.numpy as jnp
from jax import lax
from jax.experimental import pallas as pl
from jax.experimental.pallas import tpu as pltpu
from jax.experimental.pallas import tpu_sc as plsc

B, L = 1024, 32
VOCAB = 100000
EMB, HID, MEM = 64, 128, 64
NV = 1000
NSLOTS, KTOP = 256, 16
MAXIT, RELAX, DECAY = 8, 0.5, 0.997
HIDDEN = 256
TW = 16
PADW = 128

NC, NS = 2, 16
NW = NC * NS
TPW = TW * B // NW
KPW = B // NW
CHUNK = 128
NCH = TPW // CHUNK


def _softplus(x):
    return jnp.where(x > 0, x + jnp.log1p(jnp.exp(-x)), jnp.log1p(jnp.exp(x)))


@functools.cache
def _sc_kernels():
    mesh = plsc.VectorSubcoreMesh(core_axis_name="c", subcore_axis_name="s",
                                  num_cores=NC, num_subcores=NS)

    @functools.partial(
        pl.kernel,
        out_type=(jax.ShapeDtypeStruct((TW * B, PADW), jnp.float32),
                  jax.ShapeDtypeStruct((B, PADW), jnp.float32)),
        mesh=mesh,
        compiler_params=pltpu.CompilerParams(use_tc_tiling_on_sc=False),
        scratch_types=(pltpu.VMEM((NCH, CHUNK), jnp.int32),
                       pltpu.VMEM((TPW, EMB), jnp.float32),
                       pltpu.VMEM((KPW,), jnp.int32),
                       pltpu.VMEM((KPW, MEM), jnp.float32),
                       pltpu.SemaphoreType.DMA),
    )
    def _embed_gather(emb_hbm, tok_hbm, keyw_hbm, kid_hbm, x_out, kv_out,
                      tok_v, xr_v, kid_v, kr_v, sem):
        wid = lax.axis_index("s") * NC + lax.axis_index("c")
        pltpu.sync_copy(tok_hbm.at[pl.ds(wid * NCH, NCH)], tok_v)
        pltpu.sync_copy(kid_hbm.at[pl.ds(wid * KPW, KPW)], kid_v)
        cps = [pltpu.async_copy(emb_hbm.at[tok_v.at[j]],
                                xr_v.at[pl.ds(j * CHUNK, CHUNK)], sem)
               for j in range(NCH)]
        cps.append(pltpu.async_copy(keyw_hbm.at[kid_v], kr_v, sem))
        for cp in cps:
            cp.wait()
        pltpu.sync_copy(xr_v, x_out.at[pl.ds(wid * TPW, TPW), pl.ds(0, EMB)])
        pltpu.sync_copy(kr_v, kv_out.at[pl.ds(wid * KPW, KPW), pl.ds(0, MEM)])

    @functools.partial(
        pl.kernel,
        out_type=jax.ShapeDtypeStruct((NC, TW * NSLOTS, PADW), jnp.float32),
        mesh=mesh,
        compiler_params=pltpu.CompilerParams(use_tc_tiling_on_sc=False),
        scratch_types=(pltpu.VMEM((NCH, CHUNK), jnp.int32),
                       pltpu.VMEM((TPW, PADW), jnp.float32),
                       pltpu.VMEM_SHARED((TW * NSLOTS, PADW), jnp.float32),
                       pltpu.SemaphoreType.DMA),
    )
    def _scatter(rows_hbm, idx_hbm, zeros_hbm, s_out, idx_v, rows_v, shared,
                 sem):
        cid = lax.axis_index("c")
        sid = lax.axis_index("s")
        wid = sid * NC + cid
        cp_i = pltpu.async_copy(idx_hbm.at[pl.ds(wid * NCH, NCH)], idx_v, sem)
        cp_r = pltpu.async_copy(rows_hbm.at[pl.ds(wid * TPW, TPW)], rows_v,
                                sem)
        pltpu.sync_copy(zeros_hbm, shared.at[pl.ds(sid * NSLOTS, NSLOTS)])
        plsc.subcore_barrier()
        cp_i.wait()
        cp_r.wait()
        for j in range(NCH):
            pltpu.sync_copy(rows_v.at[pl.ds(j * CHUNK, CHUNK)],
                            shared.at[idx_v.at[j]], add=True)
        plsc.subcore_barrier()
        pltpu.sync_copy(shared.at[pl.ds(sid * NSLOTS, NSLOTS)],
                        s_out.at[cid, pl.ds(sid * NSLOTS, NSLOTS)])

    @functools.partial(
        pl.kernel,
        out_type=jax.ShapeDtypeStruct((KTOP * B, PADW), jnp.float32),
        mesh=mesh,
        compiler_params=pltpu.CompilerParams(use_tc_tiling_on_sc=False),
        scratch_types=(pltpu.VMEM((NCH, CHUNK), jnp.int32),
                       pltpu.VMEM((TPW, PADW), jnp.float32),
                       pltpu.SemaphoreType.DMA),
    )
    def _qgather(m_hbm, qidx_hbm, out, idx_v, rows_v, sem):
        wid = lax.axis_index("s") * NC + lax.axis_index("c")
        pltpu.sync_copy(qidx_hbm.at[pl.ds(wid * NCH, NCH)], idx_v)
        cps = [pltpu.async_copy(m_hbm.at[idx_v.at[j]],
                                rows_v.at[pl.ds(j * CHUNK, CHUNK)], sem)
               for j in range(NCH)]
        for cp in cps:
            cp.wait()
        pltpu.sync_copy(rows_v, out.at[pl.ds(wid * TPW, TPW)])

    return _embed_gather, _scatter, _qgather


def _mega_body(x_ref, kv_ref, wp_ref, qp_ref, w1_ref, b1_ref, w2_ref, b2_ref,
               wx_ref, uh_ref, bl_ref, tau_ref, h0_ref, z2v_ref, z2vb_ref,
               slota_ref, wv_ref, widx_ref, qidx_ref, w_ref, d_ref):
    f32 = jnp.float32

    def dot(a, b):
        return lax.dot_general(a, b, (((1,), (0,)), ((), ())),
                               preferred_element_type=f32)

    def dot_t(a, b):
        return lax.dot_general(a, b, (((1,), (1,)), ((), ())),
                               preferred_element_type=f32)

    kv = kv_ref[:, 0:MEM]
    kv = kv / jnp.maximum(jnp.sqrt(jnp.sum(kv * kv, axis=1, keepdims=True)),
                          1e-12)
    sim = dot_t(kv, slota_ref[...])

    iota_s = lax.broadcasted_iota(jnp.int32, (B, NSLOTS), 1)
    simm = sim
    tv, ti = [], []
    for _ in range(KTOP):
        m = jnp.max(simm, axis=1, keepdims=True)
        idx = jnp.min(jnp.where(simm == m, iota_s, NSLOTS), axis=1,
                      keepdims=True)
        tv.append(m)
        ti.append(idx)
        simm = jnp.where(iota_s == idx, -jnp.inf, simm)
    topv = jnp.concatenate(tv, axis=1)
    topi = jnp.concatenate(ti, axis=1)
    e = jnp.exp(topv - topv[:, 0:1])
    w = e / jnp.sum(e, axis=1, keepdims=True)

    w1a = w1_ref[0:HID, :]
    w1b = w1_ref[HID:HID + EMB, :]
    b1 = b1_ref[...]
    w2 = w2_ref[...]
    b2 = b2_ref[...]
    wx = wx_ref[...]
    uh = uh_ref[...]
    bl = bl_ref[...]
    tau = _softplus(tau_ref[...]) + 1.0
    z1 = z2v_ref[0:HID, :]
    z2 = z2v_ref[HID:2 * HID, :]
    z2vb = z2vb_ref[...]
    wp = wp_ref[...]

    BH = B // 2
    za = jnp.zeros((BH, HID), f32)
    zb = jnp.zeros((BH, HID), f32)
    ha = jnp.broadcast_to(h0_ref[...], (BH, HID))
    hb = ha
    vwa = jnp.zeros((BH, MEM), f32)
    vwb = jnp.zeros((BH, MEM), f32)
    wpa = wp[0:BH]
    wpb = wp[BH:B]
    for t in range(TW):
        xa = x_ref[t, 0:BH, 0:EMB]
        xb = x_ref[t, BH:B, 0:EMB]
        xwa = dot(xa, w1b) + b1
        xwb = dot(xb, w1b) + b1

        def body(i, carry, xwa=xwa, xwb=xwb):
            pa, pb = carry
            fa = pa + dot(jnp.tanh(dot(pa, w1a) + xwa), w2) + b2
            fb = pb + dot(jnp.tanh(dot(pb, w1a) + xwb), w2) + b2
            return ((1.0 - RELAX) * pa + RELAX * fa,
                    (1.0 - RELAX) * pb + RELAX * fb)

        za, zb = lax.fori_loop(0, MAXIT, body, (za, zb))
        pra = jnp.tanh(dot(za, wx) + dot(ha, uh) + bl)
        prb = jnp.tanh(dot(zb, wx) + dot(hb, uh) + bl)
        ha = ha + (pra - ha) / tau
        hb = hb + (prb - hb) / tau
        va = dot(za, z1) + dot(ha, z2) + z2vb
        vb = dot(zb, z1) + dot(hb, z2) + z2vb
        va = va / jnp.maximum(
            jnp.sqrt(jnp.sum(va * va, axis=1, keepdims=True)), 1e-12)
        vb = vb / jnp.maximum(
            jnp.sqrt(jnp.sum(vb * vb, axis=1, keepdims=True)), 1e-12)
        vwa = jnp.where(wpa == t, va, vwa)
        vwb = jnp.where(wpb == t, vb, vwb)
    vw = jnp.concatenate([vwa, vwb], axis=0)

    w_ref[...] = jnp.transpose(w)
    widx_ref[...] = jnp.transpose(wp * NSLOTS + topi)
    qidx_ref[...] = jnp.transpose(
        jnp.minimum(qp_ref[...], TW - 1) * NSLOTS + topi)
    pad = jnp.zeros((B, PADW - MEM), f32)
    for k in range(KTOP):
        wv_ref[k] = jnp.concatenate([w[:, k:k + 1] * vw, pad], axis=1)
    iota_t = lax.broadcasted_iota(jnp.int32, (B, TW), 1)
    cnt = jnp.sum((wp == iota_t).astype(f32), axis=0, keepdims=True)
    dd = jnp.where(cnt > 0, f32(DECAY), f32(1.0))
    d_ref[...] = jnp.pad(dd, ((0, 7), (0, PADW - TW)))


def _mstack_body(s_ref, d_ref, out_ref):
    s = s_ref[0] + s_ref[1]
    dv = d_ref[0:1, 0:TW]
    m = jnp.zeros((NSLOTS, PADW), jnp.float32)
    for t in range(TW):
        m = dv[0:1, t:t + 1] * m + s[t]
        out_ref[t] = m


def _final_body(m_ref, w_ref, vemb_ref, lsr_ref, out_ref):
    w = jnp.transpose(w_ref[...])
    vm = jnp.zeros((B, MEM), jnp.float32)
    for k in range(KTOP):
        vm = vm + w[:, k:k + 1] * m_ref[k, :, 0:MEM]
    vm = vm / jnp.maximum(jnp.sqrt(jnp.sum(vm * vm, axis=1, keepdims=True)),
                          1e-12)
    vp = vemb_ref[...]
    vp = vp / jnp.maximum(jnp.sqrt(jnp.sum(vp * vp, axis=1, keepdims=True)),
                          1e-12)
    scale = _softplus(lsr_ref[...]) + 1e-3
    out_ref[...] = scale * lax.dot_general(
        vm, vp, (((1,), (1,)), ((), ())), preferred_element_type=jnp.float32)


def kernel(tokens, key_ids, write_pos, query_pos, value_ids, embed_W,
           key_embed_W, value_embed_W, W1, b1, W2, b2, Wx, Uh, bl, tau_raw,
           h0, z2v_W, z2v_b, logit_scale_raw, slotA):
    i32 = jnp.int32
    f32 = jnp.float32
    embed_gather, scatter, qgather = _sc_kernels()
    tok2d = tokens[:, :TW].astype(i32).T.reshape(NW * NCH, CHUNK)
    x_rows, kv_pad = embed_gather(embed_W, tok2d, key_embed_W,
                                  key_ids.astype(i32))
    x3 = x_rows.reshape(TW, B, PADW)

    mega = pl.pallas_call(_mega_body, out_shape=(
        jax.ShapeDtypeStruct((KTOP, B, PADW), f32),
        jax.ShapeDtypeStruct((KTOP, B), i32),
        jax.ShapeDtypeStruct((KTOP, B), i32),
        jax.ShapeDtypeStruct((KTOP, B), f32),
        jax.ShapeDtypeStruct((8, PADW), f32),
    ))
    wv, widx, qidx, wts, dpad = mega(
        x3, kv_pad, write_pos.astype(i32).reshape(B, 1),
        query_pos.astype(i32).reshape(B, 1), W1, b1.reshape(1, -1), W2,
        b2.reshape(1, -1), Wx, Uh, bl.reshape(1, -1), tau_raw.reshape(1, -1),
        h0.reshape(1, -1), z2v_W, z2v_b.reshape(1, -1), slotA)

    wv_rows = wv.reshape(KTOP * B, PADW)
    widx2d = widx.reshape(NW * NCH, CHUNK)
    zeros = jnp.zeros((NSLOTS, PADW), f32)
    s2 = scatter(wv_rows, widx2d, zeros)

    mstack = pl.pallas_call(_mstack_body, out_shape=jax.ShapeDtypeStruct(
        (TW, NSLOTS, PADW), f32))
    mflat = mstack(s2.reshape(NC, TW, NSLOTS, PADW), dpad)

    qidx2d = qidx.reshape(NW * NCH, CHUNK)
    mrows = qgather(mflat.reshape(TW * NSLOTS, PADW), qidx2d)

    final = pl.pallas_call(_final_body, out_shape=jax.ShapeDtypeStruct(
        (B, NV), f32))
    return final(mrows.reshape(KTOP, B, PADW), wts, value_embed_W,
                 logit_scale_raw.reshape(1, 1))


# speedup vs baseline: 27.3434x; 1.1513x over previous
import functools

import jax
import jax.numpy as jnp
from jax import lax
from jax.experimental import pallas as pl
from jax.experimental.pallas import tpu as pltpu
from jax.experimental.pallas import tpu_sc as plsc

B, L = 1024, 32
VOCAB = 100000
EMB, HID, MEM = 64, 128, 64
NV = 1000
NSLOTS, KTOP = 256, 16
MAXIT, RELAX, DECAY = 8, 0.5, 0.997
HIDDEN = 256
TW = 16
PADW = 128

NC, NS = 2, 16
NW = NC * NS
TPW = TW * B // NW
KPW = B // NW
CHUNK = 128
NCH = TPW // CHUNK


def _softplus(x):
    return jnp.where(x > 0, x + jnp.log1p(jnp.exp(-x)), jnp.log1p(jnp.exp(x)))


@functools.cache
def _sc_kernels():
    mesh = plsc.VectorSubcoreMesh(core_axis_name="c", subcore_axis_name="s",
                                  num_cores=NC, num_subcores=NS)

    @functools.partial(
        pl.kernel,
        out_type=(jax.ShapeDtypeStruct((TW * B, PADW), jnp.float32),
                  jax.ShapeDtypeStruct((B, PADW), jnp.float32)),
        mesh=mesh,
        compiler_params=pltpu.CompilerParams(use_tc_tiling_on_sc=False),
        scratch_types=(pltpu.VMEM((NCH, CHUNK), jnp.int32),
                       pltpu.VMEM((TPW, EMB), jnp.float32),
                       pltpu.VMEM((KPW,), jnp.int32),
                       pltpu.VMEM((KPW, MEM), jnp.float32),
                       pltpu.SemaphoreType.DMA),
    )
    def _embed_gather(emb_hbm, tok_hbm, keyw_hbm, kid_hbm, x_out, kv_out,
                      tok_v, xr_v, kid_v, kr_v, sem):
        wid = lax.axis_index("s") * NC + lax.axis_index("c")
        pltpu.sync_copy(tok_hbm.at[pl.ds(wid * NCH, NCH)], tok_v)
        pltpu.sync_copy(kid_hbm.at[pl.ds(wid * KPW, KPW)], kid_v)
        cps = [pltpu.async_copy(emb_hbm.at[tok_v.at[j]],
                                xr_v.at[pl.ds(j * CHUNK, CHUNK)], sem)
               for j in range(NCH)]
        cps.append(pltpu.async_copy(keyw_hbm.at[kid_v], kr_v, sem))
        for cp in cps:
            cp.wait()
        pltpu.sync_copy(xr_v, x_out.at[pl.ds(wid * TPW, TPW), pl.ds(0, EMB)])
        pltpu.sync_copy(kr_v, kv_out.at[pl.ds(wid * KPW, KPW), pl.ds(0, MEM)])

    @functools.partial(
        pl.kernel,
        out_type=jax.ShapeDtypeStruct((NC, TW * NSLOTS, PADW), jnp.float32),
        mesh=mesh,
        compiler_params=pltpu.CompilerParams(use_tc_tiling_on_sc=False),
        scratch_types=(pltpu.VMEM((NCH, CHUNK), jnp.int32),
                       pltpu.VMEM((TPW, PADW), jnp.float32),
                       pltpu.VMEM_SHARED((TW * NSLOTS, PADW), jnp.float32),
                       pltpu.SemaphoreType.DMA),
    )
    def _scatter(rows_hbm, idx_hbm, zeros_hbm, s_out, idx_v, rows_v, shared,
                 sem):
        cid = lax.axis_index("c")
        sid = lax.axis_index("s")
        wid = sid * NC + cid
        cp_i = pltpu.async_copy(idx_hbm.at[pl.ds(wid * NCH, NCH)], idx_v, sem)
        cp_r = pltpu.async_copy(rows_hbm.at[pl.ds(wid * TPW, TPW)], rows_v,
                                sem)
        pltpu.sync_copy(zeros_hbm, shared.at[pl.ds(sid * NSLOTS, NSLOTS)])
        plsc.subcore_barrier()
        cp_i.wait()
        cp_r.wait()
        for j in range(NCH):
            pltpu.sync_copy(rows_v.at[pl.ds(j * CHUNK, CHUNK)],
                            shared.at[idx_v.at[j]], add=True)
        plsc.subcore_barrier()
        pltpu.sync_copy(shared.at[pl.ds(sid * NSLOTS, NSLOTS)],
                        s_out.at[cid, pl.ds(sid * NSLOTS, NSLOTS)])

    @functools.partial(
        pl.kernel,
        out_type=jax.ShapeDtypeStruct((KTOP * B, PADW), jnp.float32),
        mesh=mesh,
        compiler_params=pltpu.CompilerParams(use_tc_tiling_on_sc=False),
        scratch_types=(pltpu.VMEM((NCH, CHUNK), jnp.int32),
                       pltpu.VMEM((TPW, PADW), jnp.float32),
                       pltpu.SemaphoreType.DMA),
    )
    def _qgather(m_hbm, qidx_hbm, out, idx_v, rows_v, sem):
        wid = lax.axis_index("s") * NC + lax.axis_index("c")
        pltpu.sync_copy(qidx_hbm.at[pl.ds(wid * NCH, NCH)], idx_v)
        cps = [pltpu.async_copy(m_hbm.at[idx_v.at[j]],
                                rows_v.at[pl.ds(j * CHUNK, CHUNK)], sem)
               for j in range(NCH)]
        for cp in cps:
            cp.wait()
        pltpu.sync_copy(rows_v, out.at[pl.ds(wid * TPW, TPW)])

    return _embed_gather, _scatter, _qgather


def _mega_body(x_ref, kv_ref, wp_ref, qp_ref, w1_ref, b1_ref, w2_ref, b2_ref,
               wx_ref, uh_ref, bl_ref, tau_ref, h0_ref, z2v_ref, z2vb_ref,
               slota_ref, wv_ref, widx_ref, qidx_ref, w_ref, d_ref):
    f32 = jnp.float32

    def dot(a, b):
        return lax.dot_general(a, b, (((1,), (0,)), ((), ())),
                               preferred_element_type=f32)

    def dot_t(a, b):
        return lax.dot_general(a, b, (((1,), (1,)), ((), ())),
                               preferred_element_type=f32)

    kv = kv_ref[:, 0:MEM]
    kv = kv / jnp.maximum(jnp.sqrt(jnp.sum(kv * kv, axis=1, keepdims=True)),
                          1e-12)
    sim = dot_t(kv, slota_ref[...])

    iota_s = lax.broadcasted_iota(jnp.int32, (B, NSLOTS), 1)
    simm = sim
    tv, ti = [], []
    for _ in range(KTOP):
        m = jnp.max(simm, axis=1, keepdims=True)
        idx = jnp.min(jnp.where(simm == m, iota_s, NSLOTS), axis=1,
                      keepdims=True)
        tv.append(m)
        ti.append(idx)
        simm = jnp.where(iota_s == idx, -jnp.inf, simm)
    topv = jnp.concatenate(tv, axis=1)
    topi = jnp.concatenate(ti, axis=1)
    e = jnp.exp(topv - topv[:, 0:1])
    w = e / jnp.sum(e, axis=1, keepdims=True)

    w1a = w1_ref[0:HID, :]
    w1b = w1_ref[HID:HID + EMB, :]
    b1 = b1_ref[...]
    w2 = w2_ref[...]
    b2 = b2_ref[...]
    wx = wx_ref[...]
    uh = uh_ref[...]
    bl = bl_ref[...]
    tau = _softplus(tau_ref[...]) + 1.0
    z1 = z2v_ref[0:HID, :]
    z2 = z2v_ref[HID:2 * HID, :]
    z2vb = z2vb_ref[...]
    wp = wp_ref[...]

    BH = B // 2
    za = jnp.zeros((BH, HID), f32)
    zb = jnp.zeros((BH, HID), f32)
    ha = jnp.broadcast_to(h0_ref[...], (BH, HID))
    hb = ha
    vwa = jnp.zeros((BH, MEM), f32)
    vwb = jnp.zeros((BH, MEM), f32)
    wpa = wp[0:BH]
    wpb = wp[BH:B]
    for t in range(TW):
        xa = x_ref[t, 0:BH, 0:EMB]
        xb = x_ref[t, BH:B, 0:EMB]
        xwa = dot(xa, w1b) + b1
        xwb = dot(xb, w1b) + b1

        def body(i, carry, xwa=xwa, xwb=xwb):
            pa, pb = carry
            fa = pa + dot(jnp.tanh(dot(pa, w1a) + xwa), w2) + b2
            fb = pb + dot(jnp.tanh(dot(pb, w1a) + xwb), w2) + b2
            return ((1.0 - RELAX) * pa + RELAX * fa,
                    (1.0 - RELAX) * pb + RELAX * fb)

        za, zb = lax.fori_loop(0, MAXIT, body, (za, zb), unroll=4)
        pra = jnp.tanh(dot(za, wx) + dot(ha, uh) + bl)
        prb = jnp.tanh(dot(zb, wx) + dot(hb, uh) + bl)
        ha = ha + (pra - ha) / tau
        hb = hb + (prb - hb) / tau
        va = dot(za, z1) + dot(ha, z2) + z2vb
        vb = dot(zb, z1) + dot(hb, z2) + z2vb
        va = va / jnp.maximum(
            jnp.sqrt(jnp.sum(va * va, axis=1, keepdims=True)), 1e-12)
        vb = vb / jnp.maximum(
            jnp.sqrt(jnp.sum(vb * vb, axis=1, keepdims=True)), 1e-12)
        vwa = jnp.where(wpa == t, va, vwa)
        vwb = jnp.where(wpb == t, vb, vwb)
    vw = jnp.concatenate([vwa, vwb], axis=0)

    w_ref[...] = jnp.transpose(w)
    widx_ref[...] = jnp.transpose(wp * NSLOTS + topi)
    qidx_ref[...] = jnp.transpose(
        jnp.minimum(qp_ref[...], TW - 1) * NSLOTS + topi)
    pad = jnp.zeros((B, PADW - MEM), f32)
    for k in range(KTOP):
        wv_ref[k] = jnp.concatenate([w[:, k:k + 1] * vw, pad], axis=1)
    iota_t = lax.broadcasted_iota(jnp.int32, (B, TW), 1)
    cnt = jnp.sum((wp == iota_t).astype(f32), axis=0, keepdims=True)
    dd = jnp.where(cnt > 0, f32(DECAY), f32(1.0))
    d_ref[...] = jnp.pad(dd, ((0, 7), (0, PADW - TW)))


def _mstack_body(s_ref, d_ref, out_ref):
    s = s_ref[0] + s_ref[1]
    dv = d_ref[0:1, 0:TW]
    m = jnp.zeros((NSLOTS, PADW), jnp.float32)
    for t in range(TW):
        m = dv[0:1, t:t + 1] * m + s[t]
        out_ref[t] = m


def _final_body(m_ref, w_ref, vembt_ref, lsr_ref, out_ref):
    w = jnp.transpose(w_ref[...])
    vm = jnp.zeros((B, MEM), jnp.float32)
    for k in range(KTOP):
        vm = vm + w[:, k:k + 1] * m_ref[k, :, 0:MEM]
    vm = vm / jnp.maximum(jnp.sqrt(jnp.sum(vm * vm, axis=1, keepdims=True)),
                          1e-12)
    vp = vembt_ref[...]
    vp = vp / jnp.maximum(jnp.sqrt(jnp.sum(vp * vp, axis=0, keepdims=True)),
                          1e-12)
    scale = _softplus(lsr_ref[...]) + 1e-3
    out_ref[...] = scale * lax.dot_general(
        vm, vp, (((1,), (0,)), ((), ())), preferred_element_type=jnp.float32)


def kernel(tokens, key_ids, write_pos, query_pos, value_ids, embed_W,
           key_embed_W, value_embed_W, W1, b1, W2, b2, Wx, Uh, bl, tau_raw,
           h0, z2v_W, z2v_b, logit_scale_raw, slotA):
    i32 = jnp.int32
    f32 = jnp.float32
    embed_gather, scatter, qgather = _sc_kernels()
    tok2d = tokens[:, :TW].astype(i32).T.reshape(NW * NCH, CHUNK)
    x_rows, kv_pad = embed_gather(embed_W, tok2d, key_embed_W,
                                  key_ids.astype(i32))
    x3 = x_rows.reshape(TW, B, PADW)

    mega = pl.pallas_call(_mega_body, out_shape=(
        jax.ShapeDtypeStruct((KTOP, B, PADW), f32),
        jax.ShapeDtypeStruct((KTOP, B), i32),
        jax.ShapeDtypeStruct((KTOP, B), i32),
        jax.ShapeDtypeStruct((KTOP, B), f32),
        jax.ShapeDtypeStruct((8, PADW), f32),
    ))
    wv, widx, qidx, wts, dpad = mega(
        x3, kv_pad, write_pos.astype(i32).reshape(B, 1),
        query_pos.astype(i32).reshape(B, 1), W1, b1.reshape(1, -1), W2,
        b2.reshape(1, -1), Wx, Uh, bl.reshape(1, -1), tau_raw.reshape(1, -1),
        h0.reshape(1, -1), z2v_W, z2v_b.reshape(1, -1), slotA)

    wv_rows = wv.reshape(KTOP * B, PADW)
    widx2d = widx.reshape(NW * NCH, CHUNK)
    zeros = jnp.zeros((NSLOTS, PADW), f32)
    s2 = scatter(wv_rows, widx2d, zeros)

    mstack = pl.pallas_call(_mstack_body, out_shape=jax.ShapeDtypeStruct(
        (TW, NSLOTS, PADW), f32))
    mflat = mstack(s2.reshape(NC, TW, NSLOTS, PADW), dpad)

    qidx2d = qidx.reshape(NW * NCH, CHUNK)
    mrows = qgather(mflat.reshape(TW * NSLOTS, PADW), qidx2d)

    final = pl.pallas_call(_final_body, out_shape=jax.ShapeDtypeStruct(
        (B, NV), f32))
    return final(mrows.reshape(KTOP, B, PADW), wts, value_embed_W.T,
                 logit_scale_raw.reshape(1, 1))


# speedup vs baseline: 30.1616x; 1.1031x over previous
import functools

import jax
import jax.numpy as jnp
from jax import lax
from jax.experimental import pallas as pl
from jax.experimental.pallas import tpu as pltpu
from jax.experimental.pallas import tpu_sc as plsc

B, L = 1024, 32
VOCAB = 100000
EMB, HID, MEM = 64, 128, 64
NV = 1000
NSLOTS, KTOP = 256, 16
MAXIT, RELAX, DECAY = 8, 0.5, 0.997
HIDDEN = 256
TW = 16
PADW = 128

NC, NS = 2, 16
NW = NC * NS
TPW = TW * B // NW
KPW = B // NW
CHUNK = 128
NCH = TPW // CHUNK


def _softplus(x):
    return jnp.where(x > 0, x + jnp.log1p(jnp.exp(-x)), jnp.log1p(jnp.exp(x)))


@functools.cache
def _sc_kernels():
    mesh = plsc.VectorSubcoreMesh(core_axis_name="c", subcore_axis_name="s",
                                  num_cores=NC, num_subcores=NS)

    @functools.partial(
        pl.kernel,
        out_type=(jax.ShapeDtypeStruct((TW * B, PADW), jnp.float32),
                  jax.ShapeDtypeStruct((B, PADW), jnp.float32)),
        mesh=mesh,
        compiler_params=pltpu.CompilerParams(use_tc_tiling_on_sc=False),
        scratch_types=(pltpu.VMEM((NCH, CHUNK), jnp.int32),
                       pltpu.VMEM((TPW, EMB), jnp.float32),
                       pltpu.VMEM((KPW,), jnp.int32),
                       pltpu.VMEM((KPW, MEM), jnp.float32),
                       pltpu.SemaphoreType.DMA),
    )
    def _embed_gather(emb_hbm, tok_hbm, keyw_hbm, kid_hbm, x_out, kv_out,
                      tok_v, xr_v, kid_v, kr_v, sem):
        wid = lax.axis_index("s") * NC + lax.axis_index("c")
        pltpu.sync_copy(tok_hbm.at[pl.ds(wid * NCH, NCH)], tok_v)
        pltpu.sync_copy(kid_hbm.at[pl.ds(wid * KPW, KPW)], kid_v)
        cps = [pltpu.async_copy(emb_hbm.at[tok_v.at[j]],
                                xr_v.at[pl.ds(j * CHUNK, CHUNK)], sem)
               for j in range(NCH)]
        cps.append(pltpu.async_copy(keyw_hbm.at[kid_v], kr_v, sem))
        for cp in cps:
            cp.wait()
        pltpu.sync_copy(xr_v, x_out.at[pl.ds(wid * TPW, TPW), pl.ds(0, EMB)])
        pltpu.sync_copy(kr_v, kv_out.at[pl.ds(wid * KPW, KPW), pl.ds(0, MEM)])

    @functools.partial(
        pl.kernel,
        out_type=jax.ShapeDtypeStruct((NC, TW * NSLOTS, PADW), jnp.float32),
        mesh=mesh,
        compiler_params=pltpu.CompilerParams(use_tc_tiling_on_sc=False),
        scratch_types=(pltpu.VMEM((NCH, CHUNK), jnp.int32),
                       pltpu.VMEM((TPW, PADW), jnp.float32),
                       pltpu.VMEM_SHARED((TW * NSLOTS, PADW), jnp.float32),
                       pltpu.SemaphoreType.DMA),
    )
    def _scatter(rows_hbm, idx_hbm, zeros_hbm, s_out, idx_v, rows_v, shared,
                 sem):
        cid = lax.axis_index("c")
        sid = lax.axis_index("s")
        wid = sid * NC + cid
        cp_i = pltpu.async_copy(idx_hbm.at[pl.ds(wid * NCH, NCH)], idx_v, sem)
        cp_r = pltpu.async_copy(rows_hbm.at[pl.ds(wid * TPW, TPW)], rows_v,
                                sem)
        pltpu.sync_copy(zeros_hbm, shared.at[pl.ds(sid * NSLOTS, NSLOTS)])
        plsc.subcore_barrier()
        cp_i.wait()
        cp_r.wait()
        for j in range(NCH):
            pltpu.sync_copy(rows_v.at[pl.ds(j * CHUNK, CHUNK)],
                            shared.at[idx_v.at[j]], add=True)
        plsc.subcore_barrier()
        pltpu.sync_copy(shared.at[pl.ds(sid * NSLOTS, NSLOTS)],
                        s_out.at[cid, pl.ds(sid * NSLOTS, NSLOTS)])

    @functools.partial(
        pl.kernel,
        out_type=jax.ShapeDtypeStruct((KTOP * B, PADW), jnp.float32),
        mesh=mesh,
        compiler_params=pltpu.CompilerParams(use_tc_tiling_on_sc=False),
        scratch_types=(pltpu.VMEM((NCH, CHUNK), jnp.int32),
                       pltpu.VMEM((TPW, PADW), jnp.float32),
                       pltpu.SemaphoreType.DMA),
    )
    def _qgather(m_hbm, qidx_hbm, out, idx_v, rows_v, sem):
        wid = lax.axis_index("s") * NC + lax.axis_index("c")
        pltpu.sync_copy(qidx_hbm.at[pl.ds(wid * NCH, NCH)], idx_v)
        cps = [pltpu.async_copy(m_hbm.at[idx_v.at[j]],
                                rows_v.at[pl.ds(j * CHUNK, CHUNK)], sem)
               for j in range(NCH)]
        for cp in cps:
            cp.wait()
        pltpu.sync_copy(rows_v, out.at[pl.ds(wid * TPW, TPW)])

    return _embed_gather, _scatter, _qgather


def _mega_body(x_ref, kv_ref, wp_ref, qp_ref, w1_ref, b1_ref, w2_ref, b2_ref,
               wx_ref, uh_ref, bl_ref, tau_ref, h0_ref, z2v_ref, z2vb_ref,
               slota_ref, wv_ref, widx_ref, qidx_ref, w_ref, d_ref):
    f32 = jnp.float32

    def dot(a, b):
        return lax.dot_general(a, b, (((1,), (0,)), ((), ())),
                               preferred_element_type=f32)

    def dot_t(a, b):
        return lax.dot_general(a, b, (((1,), (1,)), ((), ())),
                               preferred_element_type=f32)

    kv = kv_ref[:, 0:MEM]
    kv = kv / jnp.maximum(jnp.sqrt(jnp.sum(kv * kv, axis=1, keepdims=True)),
                          1e-12)
    sim = dot_t(kv, slota_ref[...])

    iota_s = lax.broadcasted_iota(jnp.int32, (B, NSLOTS), 1)
    simm = sim
    tv, ti = [], []
    for _ in range(KTOP):
        m = jnp.max(simm, axis=1, keepdims=True)
        idx = jnp.min(jnp.where(simm == m, iota_s, NSLOTS), axis=1,
                      keepdims=True)
        tv.append(m)
        ti.append(idx)
        simm = jnp.where(iota_s == idx, -jnp.inf, simm)
    topv = jnp.concatenate(tv, axis=1)
    topi = jnp.concatenate(ti, axis=1)
    e = jnp.exp(topv - topv[:, 0:1])
    w = e / jnp.sum(e, axis=1, keepdims=True)

    w1a = w1_ref[0:HID, :]
    w1b = w1_ref[HID:HID + EMB, :]
    b1 = b1_ref[...]
    w2 = w2_ref[...]
    b2 = b2_ref[...]
    wx = wx_ref[...]
    uh = uh_ref[...]
    bl = bl_ref[...]
    tau = _softplus(tau_ref[...]) + 1.0
    z1 = z2v_ref[0:HID, :]
    z2 = z2v_ref[HID:2 * HID, :]
    z2vb = z2vb_ref[...]
    wp = wp_ref[...]

    BH = B // 2
    za = jnp.zeros((BH, HID), f32)
    zb = jnp.zeros((BH, HID), f32)
    ha = jnp.broadcast_to(h0_ref[...], (BH, HID))
    hb = ha
    vwa = jnp.zeros((BH, MEM), f32)
    vwb = jnp.zeros((BH, MEM), f32)
    wpa = wp[0:BH]
    wpb = wp[BH:B]
    for t in range(TW):
        xa = x_ref[t, 0:BH, 0:EMB]
        xb = x_ref[t, BH:B, 0:EMB]
        xwa = dot(xa, w1b) + b1
        xwb = dot(xb, w1b) + b1

        def body(i, carry, xwa=xwa, xwb=xwb):
            pa, pb = carry
            fa = pa + dot(jnp.tanh(dot(pa, w1a) + xwa), w2) + b2
            fb = pb + dot(jnp.tanh(dot(pb, w1a) + xwb), w2) + b2
            return ((1.0 - RELAX) * pa + RELAX * fa,
                    (1.0 - RELAX) * pb + RELAX * fb)

        za, zb = lax.fori_loop(0, MAXIT, body, (za, zb), unroll=8)
        pra = jnp.tanh(dot(za, wx) + dot(ha, uh) + bl)
        prb = jnp.tanh(dot(zb, wx) + dot(hb, uh) + bl)
        ha = ha + (pra - ha) / tau
        hb = hb + (prb - hb) / tau
        va = dot(za, z1) + dot(ha, z2) + z2vb
        vb = dot(zb, z1) + dot(hb, z2) + z2vb
        va = va / jnp.maximum(
            jnp.sqrt(jnp.sum(va * va, axis=1, keepdims=True)), 1e-12)
        vb = vb / jnp.maximum(
            jnp.sqrt(jnp.sum(vb * vb, axis=1, keepdims=True)), 1e-12)
        vwa = jnp.where(wpa == t, va, vwa)
        vwb = jnp.where(wpb == t, vb, vwb)
    vw = jnp.concatenate([vwa, vwb], axis=0)

    w_ref[...] = jnp.transpose(w)
    widx_ref[...] = jnp.transpose(wp * NSLOTS + topi)
    qidx_ref[...] = jnp.transpose(
        jnp.minimum(qp_ref[...], TW - 1) * NSLOTS + topi)
    pad = jnp.zeros((B, PADW - MEM), f32)
    for k in range(KTOP):
        wv_ref[k] = jnp.concatenate([w[:, k:k + 1] * vw, pad], axis=1)
    iota_t = lax.broadcasted_iota(jnp.int32, (B, TW), 1)
    cnt = jnp.sum((wp == iota_t).astype(f32), axis=0, keepdims=True)
    dd = jnp.where(cnt > 0, f32(DECAY), f32(1.0))
    d_ref[...] = jnp.pad(dd, ((0, 7), (0, PADW - TW)))


def _mstack_body(s_ref, d_ref, out_ref):
    s = s_ref[0] + s_ref[1]
    dv = d_ref[0:1, 0:TW]
    m = jnp.zeros((NSLOTS, PADW), jnp.float32)
    for t in range(TW):
        m = dv[0:1, t:t + 1] * m + s[t]
        out_ref[t] = m


def _final_body(m_ref, w_ref, vembt_ref, lsr_ref, out_ref):
    w = jnp.transpose(w_ref[...])
    vm = jnp.zeros((B, MEM), jnp.float32)
    for k in range(KTOP):
        vm = vm + w[:, k:k + 1] * m_ref[k, :, 0:MEM]
    vm = vm / jnp.maximum(jnp.sqrt(jnp.sum(vm * vm, axis=1, keepdims=True)),
                          1e-12)
    vp = vembt_ref[...]
    vp = vp / jnp.maximum(jnp.sqrt(jnp.sum(vp * vp, axis=0, keepdims=True)),
                          1e-12)
    scale = _softplus(lsr_ref[...]) + 1e-3
    out_ref[...] = scale * lax.dot_general(
        vm, vp, (((1,), (0,)), ((), ())), preferred_element_type=jnp.float32)


def kernel(tokens, key_ids, write_pos, query_pos, value_ids, embed_W,
           key_embed_W, value_embed_W, W1, b1, W2, b2, Wx, Uh, bl, tau_raw,
           h0, z2v_W, z2v_b, logit_scale_raw, slotA):
    i32 = jnp.int32
    f32 = jnp.float32
    embed_gather, scatter, qgather = _sc_kernels()
    tok2d = tokens[:, :TW].astype(i32).T.reshape(NW * NCH, CHUNK)
    x_rows, kv_pad = embed_gather(embed_W, tok2d, key_embed_W,
                                  key_ids.astype(i32))
    x3 = x_rows.reshape(TW, B, PADW)

    mega = pl.pallas_call(_mega_body, out_shape=(
        jax.ShapeDtypeStruct((KTOP, B, PADW), f32),
        jax.ShapeDtypeStruct((KTOP, B), i32),
        jax.ShapeDtypeStruct((KTOP, B), i32),
        jax.ShapeDtypeStruct((KTOP, B), f32),
        jax.ShapeDtypeStruct((8, PADW), f32),
    ))
    wv, widx, qidx, wts, dpad = mega(
        x3, kv_pad, write_pos.astype(i32).reshape(B, 1),
        query_pos.astype(i32).reshape(B, 1), W1, b1.reshape(1, -1), W2,
        b2.reshape(1, -1), Wx, Uh, bl.reshape(1, -1), tau_raw.reshape(1, -1),
        h0.reshape(1, -1), z2v_W, z2v_b.reshape(1, -1), slotA)

    wv_rows = wv.reshape(KTOP * B, PADW)
    widx2d = widx.reshape(NW * NCH, CHUNK)
    zeros = jnp.zeros((NSLOTS, PADW), f32)
    s2 = scatter(wv_rows, widx2d, zeros)

    mstack = pl.pallas_call(_mstack_body, out_shape=jax.ShapeDtypeStruct(
        (TW, NSLOTS, PADW), f32))
    mflat = mstack(s2.reshape(NC, TW, NSLOTS, PADW), dpad)

    qidx2d = qidx.reshape(NW * NCH, CHUNK)
    mrows = qgather(mflat.reshape(TW * NSLOTS, PADW), qidx2d)

    final = pl.pallas_call(_final_body, out_shape=jax.ShapeDtypeStruct(
        (B, NV), f32))
    return final(mrows.reshape(KTOP, B, PADW), wts, value_embed_W.T,
                 logit_scale_raw.reshape(1, 1))


# speedup vs baseline: 30.1798x; 1.0006x over previous
import functools

import jax
import jax.numpy as jnp
from jax import lax
from jax.experimental import pallas as pl
from jax.experimental.pallas import tpu as pltpu
from jax.experimental.pallas import tpu_sc as plsc

B, L = 1024, 32
VOCAB = 100000
EMB, HID, MEM = 64, 128, 64
NV = 1000
NSLOTS, KTOP = 256, 16
MAXIT, RELAX, DECAY = 8, 0.5, 0.997
HIDDEN = 256
TW = 16
PADW = 128

NC, NS = 2, 16
NW = NC * NS
TPW = TW * B // NW
KPW = B // NW
CHUNK = 128
NCH = TPW // CHUNK


def _softplus(x):
    return jnp.where(x > 0, x + jnp.log1p(jnp.exp(-x)), jnp.log1p(jnp.exp(x)))


@functools.cache
def _sc_kernels():
    mesh = plsc.VectorSubcoreMesh(core_axis_name="c", subcore_axis_name="s",
                                  num_cores=NC, num_subcores=NS)

    @functools.partial(
        pl.kernel,
        out_type=(jax.ShapeDtypeStruct((TW * B, PADW), jnp.float32),
                  jax.ShapeDtypeStruct((B, PADW), jnp.float32)),
        mesh=mesh,
        compiler_params=pltpu.CompilerParams(use_tc_tiling_on_sc=False),
        scratch_types=(pltpu.VMEM((NCH, CHUNK), jnp.int32),
                       pltpu.VMEM((TPW, EMB), jnp.float32),
                       pltpu.VMEM((KPW,), jnp.int32),
                       pltpu.VMEM((KPW, MEM), jnp.float32),
                       pltpu.SemaphoreType.DMA),
    )
    def _embed_gather(emb_hbm, tok_hbm, keyw_hbm, kid_hbm, x_out, kv_out,
                      tok_v, xr_v, kid_v, kr_v, sem):
        wid = lax.axis_index("s") * NC + lax.axis_index("c")
        pltpu.sync_copy(tok_hbm.at[pl.ds(wid * NCH, NCH)], tok_v)
        pltpu.sync_copy(kid_hbm.at[pl.ds(wid * KPW, KPW)], kid_v)
        cps = [pltpu.async_copy(emb_hbm.at[tok_v.at[j]],
                                xr_v.at[pl.ds(j * CHUNK, CHUNK)], sem)
               for j in range(NCH)]
        cps.append(pltpu.async_copy(keyw_hbm.at[kid_v], kr_v, sem))
        for cp in cps:
            cp.wait()
        pltpu.sync_copy(xr_v, x_out.at[pl.ds(wid * TPW, TPW), pl.ds(0, EMB)])
        pltpu.sync_copy(kr_v, kv_out.at[pl.ds(wid * KPW, KPW), pl.ds(0, MEM)])

    @functools.partial(
        pl.kernel,
        out_type=jax.ShapeDtypeStruct((NC, TW * NSLOTS, PADW), jnp.float32),
        mesh=mesh,
        compiler_params=pltpu.CompilerParams(use_tc_tiling_on_sc=False),
        scratch_types=(pltpu.VMEM((NCH, CHUNK), jnp.int32),
                       pltpu.VMEM((TPW, PADW), jnp.float32),
                       pltpu.VMEM_SHARED((TW * NSLOTS, PADW), jnp.float32),
                       pltpu.SemaphoreType.DMA),
    )
    def _scatter(rows_hbm, idx_hbm, zeros_hbm, s_out, idx_v, rows_v, shared,
                 sem):
        cid = lax.axis_index("c")
        sid = lax.axis_index("s")
        wid = sid * NC + cid
        cp_i = pltpu.async_copy(idx_hbm.at[pl.ds(wid * NCH, NCH)], idx_v, sem)
        cp_r = pltpu.async_copy(rows_hbm.at[pl.ds(wid * TPW, TPW)], rows_v,
                                sem)
        pltpu.sync_copy(zeros_hbm, shared.at[pl.ds(sid * NSLOTS, NSLOTS)])
        plsc.subcore_barrier()
        cp_i.wait()
        cp_r.wait()
        for j in range(NCH):
            pltpu.sync_copy(rows_v.at[pl.ds(j * CHUNK, CHUNK)],
                            shared.at[idx_v.at[j]], add=True)
        plsc.subcore_barrier()
        pltpu.sync_copy(shared.at[pl.ds(sid * NSLOTS, NSLOTS)],
                        s_out.at[cid, pl.ds(sid * NSLOTS, NSLOTS)])

    @functools.partial(
        pl.kernel,
        out_type=jax.ShapeDtypeStruct((KTOP * B, PADW), jnp.float32),
        mesh=mesh,
        compiler_params=pltpu.CompilerParams(use_tc_tiling_on_sc=False),
        scratch_types=(pltpu.VMEM((NCH, CHUNK), jnp.int32),
                       pltpu.VMEM((TPW, PADW), jnp.float32),
                       pltpu.SemaphoreType.DMA),
    )
    def _qgather(m_hbm, qidx_hbm, out, idx_v, rows_v, sem):
        wid = lax.axis_index("s") * NC + lax.axis_index("c")
        pltpu.sync_copy(qidx_hbm.at[pl.ds(wid * NCH, NCH)], idx_v)
        cps = [pltpu.async_copy(m_hbm.at[idx_v.at[j]],
                                rows_v.at[pl.ds(j * CHUNK, CHUNK)], sem)
               for j in range(NCH)]
        for cp in cps:
            cp.wait()
        pltpu.sync_copy(rows_v, out.at[pl.ds(wid * TPW, TPW)])

    return _embed_gather, _scatter, _qgather


def _mega_body(x_ref, kv_ref, wp_ref, qp_ref, w1_ref, b1_ref, w2_ref, b2_ref,
               wx_ref, uh_ref, bl_ref, tau_ref, h0_ref, z2v_ref, z2vb_ref,
               slota_ref, wv_ref, widx_ref, qidx_ref, w_ref, d_ref):
    f32 = jnp.float32

    def dot(a, b):
        return lax.dot_general(a, b, (((1,), (0,)), ((), ())),
                               preferred_element_type=f32)

    def dot_t(a, b):
        return lax.dot_general(a, b, (((1,), (1,)), ((), ())),
                               preferred_element_type=f32)

    kv = kv_ref[:, 0:MEM]
    kv = kv / jnp.maximum(jnp.sqrt(jnp.sum(kv * kv, axis=1, keepdims=True)),
                          1e-12)
    sim = dot_t(kv, slota_ref[...])

    iota_s = lax.broadcasted_iota(jnp.int32, (B, NSLOTS), 1)
    simm = sim
    tv, ti = [], []
    for _ in range(KTOP):
        m = jnp.max(simm, axis=1, keepdims=True)
        idx = jnp.min(jnp.where(simm == m, iota_s, NSLOTS), axis=1,
                      keepdims=True)
        tv.append(m)
        ti.append(idx)
        simm = jnp.where(iota_s == idx, -jnp.inf, simm)
    topv = jnp.concatenate(tv, axis=1)
    topi = jnp.concatenate(ti, axis=1)
    e = jnp.exp(topv - topv[:, 0:1])
    w = e / jnp.sum(e, axis=1, keepdims=True)

    w1a = w1_ref[0:HID, :]
    w1b = w1_ref[HID:HID + EMB, :]
    b1 = b1_ref[...]
    w2 = w2_ref[...]
    b2 = b2_ref[...]
    wx = wx_ref[...]
    uh = uh_ref[...]
    bl = bl_ref[...]
    tau = _softplus(tau_ref[...]) + 1.0
    z1 = z2v_ref[0:HID, :]
    z2 = z2v_ref[HID:2 * HID, :]
    z2vb = z2vb_ref[...]
    wp = wp_ref[...]

    BH = B // 2
    za = jnp.zeros((BH, HID), f32)
    zb = jnp.zeros((BH, HID), f32)
    ha = jnp.broadcast_to(h0_ref[...], (BH, HID))
    hb = ha
    vwa = jnp.zeros((BH, MEM), f32)
    vwb = jnp.zeros((BH, MEM), f32)
    wpa = wp[0:BH]
    wpb = wp[BH:B]
    for t in range(TW):
        xa = x_ref[t, 0:BH, 0:EMB]
        xb = x_ref[t, BH:B, 0:EMB]
        xwa = dot(xa, w1b) + b1
        xwb = dot(xb, w1b) + b1

        def body(i, carry, xwa=xwa, xwb=xwb):
            pa, pb = carry
            fa = pa + dot(jnp.tanh(dot(pa, w1a) + xwa), w2) + b2
            fb = pb + dot(jnp.tanh(dot(pb, w1a) + xwb), w2) + b2
            return ((1.0 - RELAX) * pa + RELAX * fa,
                    (1.0 - RELAX) * pb + RELAX * fb)

        za, zb = lax.fori_loop(0, MAXIT, body, (za, zb), unroll=8)
        pra = jnp.tanh(dot(za, wx) + dot(ha, uh) + bl)
        prb = jnp.tanh(dot(zb, wx) + dot(hb, uh) + bl)
        ha = ha + (pra - ha) / tau
        hb = hb + (prb - hb) / tau
        va = dot(za, z1) + dot(ha, z2) + z2vb
        vb = dot(zb, z1) + dot(hb, z2) + z2vb
        va = va / jnp.maximum(
            jnp.sqrt(jnp.sum(va * va, axis=1, keepdims=True)), 1e-12)
        vb = vb / jnp.maximum(
            jnp.sqrt(jnp.sum(vb * vb, axis=1, keepdims=True)), 1e-12)
        vwa = jnp.where(wpa == t, va, vwa)
        vwb = jnp.where(wpb == t, vb, vwb)
    vw = jnp.concatenate([vwa, vwb], axis=0)

    w_ref[...] = jnp.transpose(w)
    widx_ref[...] = jnp.transpose(wp * NSLOTS + topi)
    qidx_ref[...] = jnp.transpose(
        jnp.minimum(qp_ref[...], TW - 1) * NSLOTS + topi)
    pad = jnp.zeros((B, PADW - MEM), f32)
    for k in range(KTOP):
        wv_ref[k] = jnp.concatenate([w[:, k:k + 1] * vw, pad], axis=1)
    iota_t = lax.broadcasted_iota(jnp.int32, (B, TW), 1)
    cnt = jnp.sum((wp == iota_t).astype(f32), axis=0, keepdims=True)
    dd = jnp.where(cnt > 0, f32(DECAY), f32(1.0))
    d_ref[...] = jnp.pad(dd, ((0, 7), (0, PADW - TW)))


def _mstack_body(s_ref, d_ref, out_ref):
    s = s_ref[0] + s_ref[1]
    dv = d_ref[0:1, 0:TW]
    m = jnp.zeros((NSLOTS, PADW), jnp.float32)
    for t in range(TW):
        m = dv[0:1, t:t + 1] * m + s[t]
        out_ref[t] = m


def _final_body(m_ref, w_ref, vembt_ref, lsr_ref, out_ref):
    w = jnp.transpose(w_ref[...])
    vm = jnp.zeros((B, MEM), jnp.float32)
    for k in range(KTOP):
        vm = vm + w[:, k:k + 1] * m_ref[k, :, 0:MEM]
    scale = _softplus(lsr_ref[...]) + 1e-3
    vm = vm * (scale / jnp.maximum(
        jnp.sqrt(jnp.sum(vm * vm, axis=1, keepdims=True)), 1e-12))
    vp = vembt_ref[...]
    vp = vp / jnp.maximum(jnp.sqrt(jnp.sum(vp * vp, axis=0, keepdims=True)),
                          1e-12)
    out_ref[...] = lax.dot_general(
        vm, vp, (((1,), (0,)), ((), ())), preferred_element_type=jnp.float32)


def kernel(tokens, key_ids, write_pos, query_pos, value_ids, embed_W,
           key_embed_W, value_embed_W, W1, b1, W2, b2, Wx, Uh, bl, tau_raw,
           h0, z2v_W, z2v_b, logit_scale_raw, slotA):
    i32 = jnp.int32
    f32 = jnp.float32
    embed_gather, scatter, qgather = _sc_kernels()
    tok2d = tokens[:, :TW].astype(i32).T.reshape(NW * NCH, CHUNK)
    x_rows, kv_pad = embed_gather(embed_W, tok2d, key_embed_W,
                                  key_ids.astype(i32))
    x3 = x_rows.reshape(TW, B, PADW)

    mega = pl.pallas_call(_mega_body, out_shape=(
        jax.ShapeDtypeStruct((KTOP, B, PADW), f32),
        jax.ShapeDtypeStruct((KTOP, B), i32),
        jax.ShapeDtypeStruct((KTOP, B), i32),
        jax.ShapeDtypeStruct((KTOP, B), f32),
        jax.ShapeDtypeStruct((8, PADW), f32),
    ))
    wv, widx, qidx, wts, dpad = mega(
        x3, kv_pad, write_pos.astype(i32).reshape(B, 1),
        query_pos.astype(i32).reshape(B, 1), W1, b1.reshape(1, -1), W2,
        b2.reshape(1, -1), Wx, Uh, bl.reshape(1, -1), tau_raw.reshape(1, -1),
        h0.reshape(1, -1), z2v_W, z2v_b.reshape(1, -1), slotA)

    wv_rows = wv.reshape(KTOP * B, PADW)
    widx2d = widx.reshape(NW * NCH, CHUNK)
    zeros = jnp.zeros((NSLOTS, PADW), f32)
    s2 = scatter(wv_rows, widx2d, zeros)

    mstack = pl.pallas_call(_mstack_body, out_shape=jax.ShapeDtypeStruct(
        (TW, NSLOTS, PADW), f32))
    mflat = mstack(s2.reshape(NC, TW, NSLOTS, PADW), dpad)

    qidx2d = qidx.reshape(NW * NCH, CHUNK)
    mrows = qgather(mflat.reshape(TW * NSLOTS, PADW), qidx2d)

    final = pl.pallas_call(_final_body, out_shape=jax.ShapeDtypeStruct(
        (B, NV), f32))
    return final(mrows.reshape(KTOP, B, PADW), wts, value_embed_W.T,
                 logit_scale_raw.reshape(1, 1))
